# Initial kernel scaffold; baseline (speedup 1.0000x reference)
import jax
import jax.numpy as jnp
from jax.experimental import pallas as pl


def kernel(hidden_states, Wg, W1, b1, W2, b2):
    raise NotImplementedError("write your pallas kernel here")


# speedup vs baseline: 1.2624x; 1.2624x over previous
import functools

import jax
import jax.numpy as jnp
from jax import lax
from jax.experimental import pallas as pl
from jax.experimental.pallas import tpu as pltpu
from jax.experimental.pallas import tpu_sc as plsc

B, S, MDIM = 1, 2048, 1024
E, KTOP, F = 8, 2, 4096
T = B * S
C = 512
NW = 32
BPW = T // NW
DUMP_BASE = E * C
DISP_ROWS = DUMP_BASE + BPW
FBLK = 2048
NF = F // FBLK


def _cumsum0(a):
    out = a
    s = 1
    while s < T:
        shifted = jnp.concatenate(
            [jnp.zeros((s, E), jnp.float32), out[: T - s, :]], axis=0)
        out = out + shifted
        s *= 2
    return out


def _router_body(x_ref, wg_ref, gp1_ref, gp2_ref, sp1_ref, sp2_ref,
                 c1_ref, c2_ref, sv_ref):
    x = x_ref[...]
    logits = jnp.dot(x, wg_ref[...], preferred_element_type=jnp.float32)
    v1 = jnp.max(logits, axis=1, keepdims=True)
    eiota = lax.broadcasted_iota(jnp.int32, (T, E), 1)
    idx1 = jnp.min(jnp.where(logits == v1, eiota, E), axis=1, keepdims=True)
    m1 = eiota == idx1
    logits2 = jnp.where(m1, jnp.float32(-jnp.inf), logits)
    v2 = jnp.max(logits2, axis=1, keepdims=True)
    idx2 = jnp.min(jnp.where(logits2 == v2, eiota, E), axis=1, keepdims=True)
    m2 = eiota == idx2
    mask1 = m1.astype(jnp.float32)
    mask2 = m2.astype(jnp.float32)
    ex = jnp.exp(logits - v1)
    gates = ex / jnp.sum(ex, axis=1, keepdims=True)
    inc1 = _cumsum0(mask1)
    inc2 = _cumsum0(mask2)
    loc1 = inc1 - mask1
    cnt1 = inc1[T - 1:T, :]
    cnt2 = inc2[T - 1:T, :]
    loc2 = inc2 - mask2 + cnt1
    cf = jnp.float32(C)
    mask1k = jnp.where(loc1 < cf, mask1, 0.0)
    mask2k = jnp.where(loc2 < cf, mask2, 0.0)
    loc1s = jnp.sum(loc1 * mask1k, axis=1, keepdims=True)
    loc2s = jnp.sum(loc2 * mask2k, axis=1, keepdims=True)
    valid1 = jnp.sum(mask1k, axis=1, keepdims=True)
    valid2 = jnp.sum(mask2k, axis=1, keepdims=True)
    g1 = jnp.sum(gates * mask1k, axis=1, keepdims=True)
    g2 = jnp.sum(gates * mask2k, axis=1, keepdims=True)
    denom = jnp.maximum(g1 + g2, jnp.float32(jnp.finfo(jnp.float32).eps))
    c1_ref[...] = g1 / denom * valid1
    c2_ref[...] = g2 / denom * valid2
    pos1 = idx1 * C + loc1s.astype(jnp.int32)
    pos2 = idx2 * C + loc2s.astype(jnp.int32)
    gp1_ref[...] = pos1
    gp2_ref[...] = pos2
    tiota = lax.broadcasted_iota(jnp.int32, (T, 1), 0)
    dump = DUMP_BASE + (tiota & (BPW - 1))
    sp1_ref[...] = jnp.where(valid1 > 0, pos1, dump)
    sp2_ref[...] = jnp.where(valid2 > 0, pos2, dump)
    ne = jnp.minimum(cnt1 + cnt2, cf)
    citota = lax.broadcasted_iota(jnp.int32, (C, E), 0).astype(jnp.float32)
    sv_ref[...] = (citota < ne).astype(jnp.float32)


def _run_router(x, wg):
    return pl.pallas_call(
        _router_body,
        out_shape=[
            jax.ShapeDtypeStruct((T, 1), jnp.int32),
            jax.ShapeDtypeStruct((T, 1), jnp.int32),
            jax.ShapeDtypeStruct((T, 1), jnp.int32),
            jax.ShapeDtypeStruct((T, 1), jnp.int32),
            jax.ShapeDtypeStruct((T, 1), jnp.float32),
            jax.ShapeDtypeStruct((T, 1), jnp.float32),
            jax.ShapeDtypeStruct((C, E), jnp.float32),
        ],
    )(x, wg)


def _make_dispatch_sc():
    mesh = plsc.VectorSubcoreMesh(core_axis_name="c", subcore_axis_name="s")

    @functools.partial(
        pl.kernel, mesh=mesh,
        out_type=jax.ShapeDtypeStruct((DISP_ROWS, MDIM), jnp.float32),
        scratch_types=[
            pltpu.VMEM((BPW,), jnp.int32),
            pltpu.VMEM((BPW,), jnp.int32),
            pltpu.VMEM((BPW, MDIM), jnp.float32),
            pltpu.SemaphoreType.DMA,
        ],
    )
    def _dispatch_sc(x_hbm, i1_hbm, i2_hbm, out_hbm, i1v, i2v, xv, sem):
        wid = lax.axis_index("s") * 2 + lax.axis_index("c")
        base = wid * BPW
        pltpu.sync_copy(x_hbm.at[pl.ds(base, BPW)], xv)
        pltpu.sync_copy(i1_hbm.at[wid], i1v)
        pltpu.sync_copy(i2_hbm.at[wid], i2v)
        pltpu.async_copy(xv, out_hbm.at[i1v], sem).wait()
        pltpu.async_copy(xv, out_hbm.at[i2v], sem).wait()

    return _dispatch_sc


def _make_combine_gather_sc():
    mesh = plsc.VectorSubcoreMesh(core_axis_name="c", subcore_axis_name="s")

    @functools.partial(
        pl.kernel, mesh=mesh,
        out_type=[
            jax.ShapeDtypeStruct((T, MDIM), jnp.float32),
            jax.ShapeDtypeStruct((T, MDIM), jnp.float32),
        ],
        scratch_types=[
            pltpu.VMEM((BPW,), jnp.int32),
            pltpu.VMEM((BPW, MDIM), jnp.float32),
            pltpu.SemaphoreType.DMA,
        ],
    )
    def _combine_gather_sc(eo_hbm, g1_hbm, g2_hbm, r1_hbm, r2_hbm,
                           iv, rv, sem):
        wid = lax.axis_index("s") * 2 + lax.axis_index("c")
        base = wid * BPW
        pltpu.sync_copy(g1_hbm.at[wid], iv)
        pltpu.async_copy(eo_hbm.at[iv], rv, sem).wait()
        pltpu.sync_copy(rv, r1_hbm.at[pl.ds(base, BPW)])
        pltpu.sync_copy(g2_hbm.at[wid], iv)
        pltpu.async_copy(eo_hbm.at[iv], rv, sem).wait()
        pltpu.sync_copy(rv, r2_hbm.at[pl.ds(base, BPW)])

    return _combine_gather_sc


def _ffn_body(sv_ref, disp_ref, w1_ref, b1_ref, w2_ref, b2_ref, eo_ref,
              dm_ref):
    f = pl.program_id(1)

    @pl.when(f == 0)
    def _():
        dm_ref[...] = jnp.where(sv_ref[0] > 0, disp_ref[...], 0.0)

    h = jnp.maximum(
        jnp.dot(dm_ref[...], w1_ref[0], preferred_element_type=jnp.float32)
        + b1_ref[0], 0.0)
    part = jnp.dot(h, w2_ref[0], preferred_element_type=jnp.float32)

    @pl.when(f == 0)
    def _():
        eo_ref[0] = part + b2_ref[0]

    @pl.when(f > 0)
    def _():
        eo_ref[0] = eo_ref[0] + part


def _run_ffn(sv3, disp, w1, b1, w2, b2):
    return pl.pallas_call(
        _ffn_body,
        grid=(E, NF),
        in_specs=[
            pl.BlockSpec((1, C, 1), lambda e, f: (e, 0, 0)),
            pl.BlockSpec((C, MDIM), lambda e, f: (e, 0)),
            pl.BlockSpec((1, MDIM, FBLK), lambda e, f: (e, 0, f)),
            pl.BlockSpec((1, 1, FBLK), lambda e, f: (e, 0, f)),
            pl.BlockSpec((1, FBLK, MDIM), lambda e, f: (e, f, 0)),
            pl.BlockSpec((1, 1, MDIM), lambda e, f: (e, 0, 0)),
        ],
        out_specs=pl.BlockSpec((1, C, MDIM), lambda e, f: (e, 0, 0)),
        out_shape=jax.ShapeDtypeStruct((E, C, MDIM), jnp.float32),
        scratch_shapes=[pltpu.VMEM((C, MDIM), jnp.float32)],
    )(sv3, disp, w1, b1.reshape(E, 1, F), w2, b2.reshape(E, 1, MDIM))


def _wt_body(c1_ref, c2_ref, r1_ref, r2_ref, o_ref):
    o_ref[...] = c1_ref[...] * r1_ref[...] + c2_ref[...] * r2_ref[...]


def _run_weight(c1, c2, r1, r2):
    return pl.pallas_call(
        _wt_body,
        out_shape=jax.ShapeDtypeStruct((T, MDIM), jnp.float32),
    )(c1, c2, r1, r2)


def kernel(hidden_states, Wg, W1, b1, W2, b2):
    x = hidden_states.reshape(T, MDIM)
    gp1, gp2, sp1, sp2, c1, c2, sv = _run_router(x, Wg)
    disp = _make_dispatch_sc()(x, sp1.reshape(NW, BPW), sp2.reshape(NW, BPW))
    sv3 = sv.T.reshape(E, C, 1)
    eo = _run_ffn(sv3, disp, W1, b1, W2, b2)
    r1, r2 = _make_combine_gather_sc()(
        eo.reshape(E * C, MDIM), gp1.reshape(NW, BPW), gp2.reshape(NW, BPW))
    out = _run_weight(c1, c2, r1, r2)
    return out.reshape(B, S, MDIM), jnp.zeros((), jnp.float32)
